# baseline (speedup 1.0000x reference)
import functools

import jax
import jax.numpy as jnp
from jax import lax
from jax.experimental import pallas as pl
from jax.experimental.pallas import tpu as pltpu
from jax.experimental.pallas import tpu_sc as plsc

_N = 10000
_E = 160000
_D = 256
_DH = _D // 2
_K = 128
_NSUB = 16
_NCORE = 2
_ACC1 = 10240
_ACC2 = 10112
_RRED = _ACC1 // _NSUB
_BLK = 1000


@functools.lru_cache(maxsize=None)
def _make_sc_agg(with_cnt, chunks, nbuf, acc_rows, k):
    n_outer = chunks // nbuf
    rows_per_sub = 624
    zps = acc_rows // _NSUB
    mesh = plsc.VectorSubcoreMesh(
        core_axis_name="c", subcore_axis_name="s",
        num_cores=_NCORE, num_subcores=_NSUB)

    nq = 2 * nbuf
    out_type = [jax.ShapeDtypeStruct((_NCORE * _N, _DH), jnp.float32)]
    scratch = [
        pltpu.VMEM((nq, 2, k), jnp.int32),
        pltpu.VMEM((nbuf, k, _DH), jnp.float32),
        pltpu.VMEM_SHARED((acc_rows, _DH), jnp.float32),
        pltpu.SemaphoreType.DMA((nbuf,)),
        pltpu.SemaphoreType.DMA((nbuf,)),
        pltpu.SemaphoreType.DMA((nq,)),
    ]
    if with_cnt:
        out_type.append(jax.ShapeDtypeStruct((_ACC1,), jnp.float32))
        out_type.append(jax.ShapeDtypeStruct((_NSUB, _ACC1), jnp.float32))
        scratch += [
            pltpu.VMEM((_ACC1,), jnp.float32),
            pltpu.VMEM((_RRED,), jnp.float32),
            pltpu.VMEM((_RRED,), jnp.float32),
        ]

    @functools.partial(pl.kernel, out_type=out_type, mesh=mesh,
                       scratch_types=scratch,
                       compiler_params=pltpu.CompilerParams(needs_layout_passes=False))
    def sc_agg(y_hbm, eidx_hbm, out_hbm, *rest):
        if with_cnt:
            (cnt_hbm, stage_hbm, idx_v, rows_v, acc, gsem, ssem, isem,
             hist_v, red_v, tmp_v) = rest
        else:
            idx_v, rows_v, acc, gsem, ssem, isem = rest
        cid = lax.axis_index("c")
        sid = lax.axis_index("s")

        def zrow(i, _):
            for j in range(_DH // 16):
                rows_v[0, i, pl.ds(j * 16, 16)] = jnp.zeros((16,), jnp.float32)
            return 0
        lax.fori_loop(0, k, zrow, 0)

        zbase = sid * zps
        for q in range(zps // k):
            pltpu.sync_copy(rows_v.at[0], acc.at[pl.ds(zbase + q * k, k)])
        ztail = zps % k
        if ztail:
            pltpu.sync_copy(rows_v.at[0].at[pl.ds(0, ztail)],
                            acc.at[pl.ds(zbase + (zps // k) * k, ztail)])

        if with_cnt:
            def hzero(i, _):
                hist_v[pl.ds(i * 16, 16)] = jnp.zeros((16,), jnp.float32)
                return 0
            lax.fori_loop(0, _ACC1 // 16, hzero, 0)

        plsc.subcore_barrier()

        cbase0 = sid * chunks

        def load_idx(g, q):
            pltpu.async_copy(eidx_hbm.at[cid, cbase0 + g], idx_v.at[q], isem.at[q])

        def wait_idx(g, q):
            pltpu.make_async_copy(eidx_hbm.at[cid, cbase0 + g], idx_v.at[q],
                                  isem.at[q]).wait()

        def start_gather(b, q):
            pltpu.async_copy(y_hbm.at[idx_v.at[q, 0]], rows_v.at[b], gsem.at[b])

        def wait_gather(b, q):
            pltpu.make_async_copy(y_hbm.at[idx_v.at[q, 0]], rows_v.at[b],
                                  gsem.at[b]).wait()

        def start_scatter(b, q):
            pltpu.async_copy(rows_v.at[b], acc.at[idx_v.at[q, 1]], ssem.at[b], add=True)

        def wait_scatter(b, q):
            pltpu.make_async_copy(rows_v.at[b], acc.at[idx_v.at[q, 1]],
                                  ssem.at[b]).wait()

        ones16 = jnp.ones((16,), jnp.float32)

        def step(g, u, first=False, do_load=True, do_gather=True):
            b, q = u % nbuf, u % nq
            wait_gather(b, q)
            start_scatter(b, q)
            if with_cnt and False:
                for j in range(k // 16):
                    di = idx_v[q, 1, pl.ds(j * 16, 16)]
                    plsc.addupdate_scatter(hist_v, [di], ones16)
            if not first:
                wait_scatter((u - 1) % nbuf, (u - 1) % nq)
            if do_load:
                load_idx(g + nq - 1, (u - 1) % nq)
            if do_gather:
                gg = g + nbuf - 1
                ug = u + nbuf - 1
                wait_idx(gg, ug % nq)
                start_gather(ug % nbuf, ug % nq)

        for gg in range(nq - 1):
            load_idx(gg, gg)
        for gg in range(nbuf - 1):
            wait_idx(gg, gg)
            start_gather(gg, gg)

        for g in range(nq):
            step(g, g, first=(g == 0))

        def outer(go, _):
            for u in range(nq):
                step(go * nq + u, u)
            return 0
        lax.fori_loop(1, chunks // nq - 1, outer, 0)

        for g in range(chunks - nq, chunks):
            step(g, g, do_load=(g + nq - 1 < chunks),
                 do_gather=(g + nbuf - 1 < chunks))

        wait_scatter((chunks - 1) % nbuf, (chunks - 1) % nq)

        if with_cnt:
            @pl.when(cid == 0)
            def _():
                pltpu.sync_copy(hist_v, stage_hbm.at[sid])

        plsc.subcore_barrier()

        wbase = sid * rows_per_sub
        pltpu.sync_copy(acc.at[pl.ds(wbase, rows_per_sub)],
                        out_hbm.at[pl.ds(cid * _N + wbase, rows_per_sub)])

        @pl.when(sid == _NSUB - 1)
        def _():
            tail = _NSUB * rows_per_sub
            pltpu.sync_copy(acc.at[pl.ds(tail, _N - tail)],
                            out_hbm.at[pl.ds(cid * _N + tail, _N - tail)])

        if with_cnt:
            @pl.when(cid == 0)
            def _():
                hbase = sid * _RRED
                pltpu.sync_copy(stage_hbm.at[0, pl.ds(hbase, _RRED)], red_v)

                def red_t(t, _):
                    pltpu.sync_copy(stage_hbm.at[t, pl.ds(hbase, _RRED)], tmp_v)
                    for j in range(_RRED // 16):
                        red_v[pl.ds(j * 16, 16)] = (red_v[pl.ds(j * 16, 16)]
                                                    + tmp_v[pl.ds(j * 16, 16)])
                    return 0
                lax.fori_loop(1, _NSUB, red_t, 0)
                pltpu.sync_copy(red_v, cnt_hbm.at[pl.ds(hbase, _RRED)])

    return sc_agg


def _mmA_body(x_ref, wl_ref, wr_ref, b_ref, y_ref, z_ref):
    xb = x_ref[...]
    dn = (((1,), (1,)), ((), ()))
    y = lax.dot_general(xb, wl_ref[...], dn, preferred_element_type=jnp.float32)
    z = lax.dot_general(xb, wr_ref[...], dn, preferred_element_type=jnp.float32)
    z_ref[...] = z + b_ref[...]
    y_ref[0] = y[:, :_DH]
    y_ref[1] = y[:, _DH:]


def _mmB_body(s_ref, c_ref, z_ref, wl_ref, wr_ref, b_ref, y_ref, z2_ref, inv_ref):
    inv = 1.0 / jnp.maximum(c_ref[...], 1.0)
    h = jnp.maximum(
        jnp.concatenate([s_ref[0], s_ref[1]], axis=1) * inv + z_ref[...], 0.0)
    dn = (((1,), (1,)), ((), ()))
    y = lax.dot_general(h, wl_ref[...], dn, preferred_element_type=jnp.float32)
    z2 = lax.dot_general(h, wr_ref[...], dn, preferred_element_type=jnp.float32)
    z2_ref[...] = z2 + b_ref[...]
    y_ref[0] = y[:, :_DH]
    y_ref[1] = y[:, _DH:]
    inv_ref[...] = inv


def _mmC_body(s_ref, z_ref, inv_ref, o_ref):
    inv = inv_ref[...]
    o_ref[...] = jnp.concatenate([s_ref[0] * inv, s_ref[1] * inv], axis=1) + z_ref[...]


def _mmA(x, W1l, W1r, b1):
    grid = (_N // _BLK,)
    return pl.pallas_call(
        _mmA_body,
        grid=grid,
        in_specs=[
            pl.BlockSpec((_BLK, _D), lambda i: (i, 0)),
            pl.BlockSpec((_D, _D), lambda i: (0, 0)),
            pl.BlockSpec((_D, _D), lambda i: (0, 0)),
            pl.BlockSpec((1, _D), lambda i: (0, 0)),
        ],
        out_specs=[
            pl.BlockSpec((2, _BLK, _DH), lambda i: (0, i, 0)),
            pl.BlockSpec((_BLK, _D), lambda i: (i, 0)),
        ],
        out_shape=[
            jax.ShapeDtypeStruct((2, _N, _DH), jnp.float32),
            jax.ShapeDtypeStruct((_N, _D), jnp.float32),
        ],
    )(x, W1l, W1r, b1.reshape(1, _D))


def _mmB(S1, cnt, z1, W2l, W2r, b2):
    grid = (_N // _BLK,)
    return pl.pallas_call(
        _mmB_body,
        grid=grid,
        in_specs=[
            pl.BlockSpec((2, _BLK, _DH), lambda i: (0, i, 0)),
            pl.BlockSpec((_BLK, 1), lambda i: (i, 0)),
            pl.BlockSpec((_BLK, _D), lambda i: (i, 0)),
            pl.BlockSpec((_D, _D), lambda i: (0, 0)),
            pl.BlockSpec((_D, _D), lambda i: (0, 0)),
            pl.BlockSpec((1, _D), lambda i: (0, 0)),
        ],
        out_specs=[
            pl.BlockSpec((2, _BLK, _DH), lambda i: (0, i, 0)),
            pl.BlockSpec((_BLK, _D), lambda i: (i, 0)),
            pl.BlockSpec((_BLK, 1), lambda i: (i, 0)),
        ],
        out_shape=[
            jax.ShapeDtypeStruct((2, _N, _DH), jnp.float32),
            jax.ShapeDtypeStruct((_N, _D), jnp.float32),
            jax.ShapeDtypeStruct((_N, 1), jnp.float32),
        ],
    )(S1, cnt, z1, W2l, W2r, b2.reshape(1, _D))


def _mmC(S2, z2, inv):
    grid = (_N // _BLK,)
    return pl.pallas_call(
        _mmC_body,
        grid=grid,
        in_specs=[
            pl.BlockSpec((2, _BLK, _DH), lambda i: (0, i, 0)),
            pl.BlockSpec((_BLK, _D), lambda i: (i, 0)),
            pl.BlockSpec((_BLK, 1), lambda i: (i, 0)),
        ],
        out_specs=pl.BlockSpec((_BLK, _D), lambda i: (i, 0)),
        out_shape=jax.ShapeDtypeStruct((_N, _D), jnp.float32),
    )(S2, z2, inv)


def _edge_chunks(edge_index, epad, k):
    pad = epad - _E
    src = jnp.concatenate([edge_index[0], jnp.zeros((pad,), jnp.int32)]).reshape(-1, k)
    dst = jnp.concatenate([edge_index[1], jnp.full((pad,), _N, jnp.int32)]).reshape(-1, k)
    lo = jnp.stack([src, dst], axis=1)
    hi = jnp.stack([src + _N, dst], axis=1)
    return jnp.stack([lo, hi])


def kernel(x, edge_index, W1l, b1, W1r, W2l, b2, W2r):
    e1 = _edge_chunks(edge_index, _NSUB * 108 * 96, 96)
    e2 = _edge_chunks(edge_index, _NSUB * 84 * 120, 120)

    y1, z1 = _mmA(x, W1l, W1r, b1)
    S1, cnt_pad, _ = _make_sc_agg(True, 108, 3, _ACC2, 96)(
        y1.reshape(_NCORE * _N, _DH), e1)
    cnt = cnt_pad[:_N].reshape(_N, 1)
    y2, z2, inv = _mmB(S1.reshape(_NCORE, _N, _DH), cnt, z1, W2l, W2r, b2)
    (S2,) = _make_sc_agg(False, 84, 3, _ACC2, 120)(y2.reshape(_NCORE * _N, _DH), e2)
    return _mmC(S2.reshape(_NCORE, _N, _DH), z2, inv)


# speedup vs baseline: 1.6693x; 1.6693x over previous
import functools

import jax
import jax.numpy as jnp
from jax import lax
from jax.experimental import pallas as pl
from jax.experimental.pallas import tpu as pltpu
from jax.experimental.pallas import tpu_sc as plsc

_N = 10000
_E = 160000
_D = 256
_DH = _D // 2
_K = 128
_NSUB = 16
_NCORE = 2
_ACC1 = 10240
_ACC2 = 10112
_RRED = _ACC1 // _NSUB
_BLK = 1000


@functools.lru_cache(maxsize=None)
def _make_sc_agg(chunks, nbuf, acc_rows, k):
    n_outer = chunks // nbuf
    rows_per_sub = 624
    zps = acc_rows // _NSUB
    mesh = plsc.VectorSubcoreMesh(
        core_axis_name="c", subcore_axis_name="s",
        num_cores=_NCORE, num_subcores=_NSUB)

    nq = 2 * nbuf
    out_type = [jax.ShapeDtypeStruct((_NCORE * _N, _DH), jnp.float32)]
    scratch = [
        pltpu.VMEM((nq, 2, k), jnp.int32),
        pltpu.VMEM((nbuf, k, _DH), jnp.float32),
        pltpu.VMEM_SHARED((acc_rows, _DH), jnp.float32),
        pltpu.SemaphoreType.DMA((nbuf,)),
        pltpu.SemaphoreType.DMA((nbuf,)),
        pltpu.SemaphoreType.DMA((nq,)),
    ]
    @functools.partial(pl.kernel, out_type=out_type, mesh=mesh,
                       scratch_types=scratch,
                       compiler_params=pltpu.CompilerParams(needs_layout_passes=False))
    def sc_agg(y_hbm, eidx_hbm, out_hbm, idx_v, rows_v, acc, gsem, ssem, isem):
        cid = lax.axis_index("c")
        sid = lax.axis_index("s")

        def zrow(i, _):
            for j in range(_DH // 16):
                rows_v[0, i, pl.ds(j * 16, 16)] = jnp.zeros((16,), jnp.float32)
            return 0
        lax.fori_loop(0, k, zrow, 0)

        zbase = sid * zps
        for q in range(zps // k):
            pltpu.sync_copy(rows_v.at[0], acc.at[pl.ds(zbase + q * k, k)])
        ztail = zps % k
        if ztail:
            pltpu.sync_copy(rows_v.at[0].at[pl.ds(0, ztail)],
                            acc.at[pl.ds(zbase + (zps // k) * k, ztail)])

        plsc.subcore_barrier()

        cbase0 = sid * chunks

        def load_idx(g, q):
            pltpu.async_copy(eidx_hbm.at[cid, cbase0 + g], idx_v.at[q], isem.at[q])

        def wait_idx(g, q):
            pltpu.make_async_copy(eidx_hbm.at[cid, cbase0 + g], idx_v.at[q],
                                  isem.at[q]).wait()

        def start_gather(b, q):
            pltpu.async_copy(y_hbm.at[idx_v.at[q, 0]], rows_v.at[b], gsem.at[b])

        def wait_gather(b, q):
            pltpu.make_async_copy(y_hbm.at[idx_v.at[q, 0]], rows_v.at[b],
                                  gsem.at[b]).wait()

        def start_scatter(b, q):
            pltpu.async_copy(rows_v.at[b], acc.at[idx_v.at[q, 1]], ssem.at[b], add=True)

        def wait_scatter(b, q):
            pltpu.make_async_copy(rows_v.at[b], acc.at[idx_v.at[q, 1]],
                                  ssem.at[b]).wait()

        def step(g, u, first=False, do_load=True, do_gather=True):
            b, q = u % nbuf, u % nq
            wait_gather(b, q)
            start_scatter(b, q)
            if not first:
                wait_scatter((u - 1) % nbuf, (u - 1) % nq)
            if do_load:
                load_idx(g + nq - 1, (u - 1) % nq)
            if do_gather:
                gg = g + nbuf - 1
                ug = u + nbuf - 1
                wait_idx(gg, ug % nq)
                start_gather(ug % nbuf, ug % nq)

        for gg in range(nq - 1):
            load_idx(gg, gg)
        for gg in range(nbuf - 1):
            wait_idx(gg, gg)
            start_gather(gg, gg)

        for g in range(nq):
            step(g, g, first=(g == 0))

        def outer(go, _):
            for u in range(nq):
                step(go * nq + u, u)
            return 0
        lax.fori_loop(1, chunks // nq - 1, outer, 0)

        for g in range(chunks - nq, chunks):
            step(g, g, do_load=(g + nq - 1 < chunks),
                 do_gather=(g + nbuf - 1 < chunks))

        wait_scatter((chunks - 1) % nbuf, (chunks - 1) % nq)

        plsc.subcore_barrier()

        wbase = sid * rows_per_sub
        pltpu.sync_copy(acc.at[pl.ds(wbase, rows_per_sub)],
                        out_hbm.at[pl.ds(cid * _N + wbase, rows_per_sub)])

        @pl.when(sid == _NSUB - 1)
        def _():
            tail = _NSUB * rows_per_sub
            pltpu.sync_copy(acc.at[pl.ds(tail, _N - tail)],
                            out_hbm.at[pl.ds(cid * _N + tail, _N - tail)])


    return sc_agg


_ECNT = 163840


@functools.lru_cache(maxsize=None)
def _make_sc_cnt():
    per_tile = _ECNT // (_NCORE * _NSUB)
    mesh = plsc.VectorSubcoreMesh(
        core_axis_name="c", subcore_axis_name="s",
        num_cores=_NCORE, num_subcores=_NSUB)

    @functools.partial(
        pl.kernel,
        out_type=jax.ShapeDtypeStruct((_NCORE, _ACC1), jnp.float32),
        mesh=mesh,
        scratch_types=[
            pltpu.VMEM((per_tile,), jnp.int32),
            pltpu.VMEM((_ACC1,), jnp.float32),
            pltpu.VMEM((_NSUB, _RRED), jnp.float32),
            pltpu.VMEM((_RRED,), jnp.float32),
            pltpu.VMEM_SHARED((_NSUB, _ACC1), jnp.float32),
        ],
        compiler_params=pltpu.CompilerParams(needs_layout_passes=False))
    def sc_cnt(dst_hbm, cnt_hbm, dstb, hist, stg, red, stage):
        cid = lax.axis_index("c")
        sid = lax.axis_index("s")
        wid = sid * _NCORE + cid
        pltpu.sync_copy(dst_hbm.at[pl.ds(wid * per_tile, per_tile)], dstb)

        def hzero(i, _):
            hist[pl.ds(i * 16, 16)] = jnp.zeros((16,), jnp.float32)
            return 0
        lax.fori_loop(0, _ACC1 // 16, hzero, 0)

        ones16 = jnp.ones((16,), jnp.float32)

        def upd(i, _):
            di = dstb[pl.ds(i * 16, 16)]
            plsc.addupdate_scatter(hist, [di], ones16)
            return 0
        lax.fori_loop(0, per_tile // 16, upd, 0)

        pltpu.sync_copy(hist, stage.at[sid])
        plsc.subcore_barrier()

        pltpu.sync_copy(stage.at[:, pl.ds(sid * _RRED, _RRED)], stg)

        def red_t(t, _):
            for j in range(_RRED // 16):
                red[pl.ds(j * 16, 16)] = (red[pl.ds(j * 16, 16)]
                                          + stg[t, pl.ds(j * 16, 16)])
            return 0
        for j in range(_RRED // 16):
            red[pl.ds(j * 16, 16)] = stg[0, pl.ds(j * 16, 16)]
        lax.fori_loop(1, _NSUB, red_t, 0)
        pltpu.sync_copy(red, cnt_hbm.at[cid, pl.ds(sid * _RRED, _RRED)])

    return sc_cnt


def _mmA_body(x_ref, wl_ref, wr_ref, b_ref, y_ref, z_ref):
    xb = x_ref[...]
    dn = (((1,), (1,)), ((), ()))
    y = lax.dot_general(xb, wl_ref[...], dn, preferred_element_type=jnp.float32)
    z = lax.dot_general(xb, wr_ref[...], dn, preferred_element_type=jnp.float32)
    z_ref[...] = z + b_ref[...]
    y_ref[0] = y[:, :_DH]
    y_ref[1] = y[:, _DH:]


def _mmB_body(s_ref, c_ref, c2_ref, z_ref, wl_ref, wr_ref, b_ref, y_ref, z2_ref, inv_ref):
    inv = 1.0 / jnp.maximum(c_ref[...] + c2_ref[...], 1.0)
    h = jnp.maximum(
        jnp.concatenate([s_ref[0], s_ref[1]], axis=1) * inv + z_ref[...], 0.0)
    dn = (((1,), (1,)), ((), ()))
    y = lax.dot_general(h, wl_ref[...], dn, preferred_element_type=jnp.float32)
    z2 = lax.dot_general(h, wr_ref[...], dn, preferred_element_type=jnp.float32)
    z2_ref[...] = z2 + b_ref[...]
    y_ref[0] = y[:, :_DH]
    y_ref[1] = y[:, _DH:]
    inv_ref[...] = inv


def _mmC_body(s_ref, z_ref, inv_ref, o_ref):
    inv = inv_ref[...]
    o_ref[...] = jnp.concatenate([s_ref[0] * inv, s_ref[1] * inv], axis=1) + z_ref[...]


def _mmA(x, W1l, W1r, b1):
    grid = (_N // _BLK,)
    return pl.pallas_call(
        _mmA_body,
        grid=grid,
        in_specs=[
            pl.BlockSpec((_BLK, _D), lambda i: (i, 0)),
            pl.BlockSpec((_D, _D), lambda i: (0, 0)),
            pl.BlockSpec((_D, _D), lambda i: (0, 0)),
            pl.BlockSpec((1, _D), lambda i: (0, 0)),
        ],
        out_specs=[
            pl.BlockSpec((2, _BLK, _DH), lambda i: (0, i, 0)),
            pl.BlockSpec((_BLK, _D), lambda i: (i, 0)),
        ],
        out_shape=[
            jax.ShapeDtypeStruct((2, _N, _DH), jnp.float32),
            jax.ShapeDtypeStruct((_N, _D), jnp.float32),
        ],
    )(x, W1l, W1r, b1.reshape(1, _D))


def _mmB(S1, cnt, cnt2, z1, W2l, W2r, b2):
    grid = (_N // _BLK,)
    return pl.pallas_call(
        _mmB_body,
        grid=grid,
        in_specs=[
            pl.BlockSpec((2, _BLK, _DH), lambda i: (0, i, 0)),
            pl.BlockSpec((_BLK, 1), lambda i: (i, 0)),
            pl.BlockSpec((_BLK, 1), lambda i: (i, 0)),
            pl.BlockSpec((_BLK, _D), lambda i: (i, 0)),
            pl.BlockSpec((_D, _D), lambda i: (0, 0)),
            pl.BlockSpec((_D, _D), lambda i: (0, 0)),
            pl.BlockSpec((1, _D), lambda i: (0, 0)),
        ],
        out_specs=[
            pl.BlockSpec((2, _BLK, _DH), lambda i: (0, i, 0)),
            pl.BlockSpec((_BLK, _D), lambda i: (i, 0)),
            pl.BlockSpec((_BLK, 1), lambda i: (i, 0)),
        ],
        out_shape=[
            jax.ShapeDtypeStruct((2, _N, _DH), jnp.float32),
            jax.ShapeDtypeStruct((_N, _D), jnp.float32),
            jax.ShapeDtypeStruct((_N, 1), jnp.float32),
        ],
    )(S1, cnt, cnt2, z1, W2l, W2r, b2.reshape(1, _D))


def _mmC(S2, z2, inv):
    grid = (_N // _BLK,)
    return pl.pallas_call(
        _mmC_body,
        grid=grid,
        in_specs=[
            pl.BlockSpec((2, _BLK, _DH), lambda i: (0, i, 0)),
            pl.BlockSpec((_BLK, _D), lambda i: (i, 0)),
            pl.BlockSpec((_BLK, 1), lambda i: (i, 0)),
        ],
        out_specs=pl.BlockSpec((_BLK, _D), lambda i: (i, 0)),
        out_shape=jax.ShapeDtypeStruct((_N, _D), jnp.float32),
    )(S2, z2, inv)


def _edge_chunks(edge_index, epad, k):
    pad = epad - _E
    src = jnp.concatenate([edge_index[0], jnp.zeros((pad,), jnp.int32)]).reshape(-1, k)
    dst = jnp.concatenate([edge_index[1], jnp.full((pad,), _N, jnp.int32)]).reshape(-1, k)
    lo = jnp.stack([src, dst], axis=1)
    hi = jnp.stack([src + _N, dst], axis=1)
    return jnp.stack([lo, hi])


def kernel(x, edge_index, W1l, b1, W1r, W2l, b2, W2r):
    e2 = _edge_chunks(edge_index, _NSUB * 84 * 120, 120)
    dst_p = jnp.concatenate(
        [edge_index[1], jnp.full((_ECNT - _E,), _N, jnp.int32)])

    cnt01 = _make_sc_cnt()(dst_p)
    y1, z1 = _mmA(x, W1l, W1r, b1)
    (S1,) = _make_sc_agg(84, 3, _ACC2, 120)(y1.reshape(_NCORE * _N, _DH), e2)
    cnt = cnt01[0, :_N].reshape(_N, 1)
    cnt2 = cnt01[1, :_N].reshape(_N, 1)
    y2, z2, inv = _mmB(S1.reshape(_NCORE, _N, _DH), cnt, cnt2, z1, W2l, W2r, b2)
    (S2,) = _make_sc_agg(84, 3, _ACC2, 120)(y2.reshape(_NCORE * _N, _DH), e2)
    return _mmC(S2.reshape(_NCORE, _N, _DH), z2, inv)


# speedup vs baseline: 1.6814x; 1.0072x over previous
import functools

import jax
import jax.numpy as jnp
from jax import lax
from jax.experimental import pallas as pl
from jax.experimental.pallas import tpu as pltpu
from jax.experimental.pallas import tpu_sc as plsc

_N = 10000
_E = 160000
_D = 256
_DH = _D // 2
_K = 128
_NSUB = 16
_NCORE = 2
_ACC1 = 10240
_ACC2 = 10112
_RRED = _ACC1 // _NSUB
_BLK = 1000


@functools.lru_cache(maxsize=None)
def _make_sc_agg(chunks, nbuf, acc_rows, k):
    n_outer = chunks // nbuf
    rows_per_sub = 624
    zps = acc_rows // _NSUB
    mesh = plsc.VectorSubcoreMesh(
        core_axis_name="c", subcore_axis_name="s",
        num_cores=_NCORE, num_subcores=_NSUB)

    nq = 2 * nbuf
    out_type = [jax.ShapeDtypeStruct((_NCORE * _N, _DH), jnp.float32)]
    scratch = [
        pltpu.VMEM((nq, 2, k), jnp.int32),
        pltpu.VMEM((nbuf, k, _DH), jnp.float32),
        pltpu.VMEM_SHARED((acc_rows, _DH), jnp.float32),
        pltpu.SemaphoreType.DMA((nbuf,)),
        pltpu.SemaphoreType.DMA((nbuf,)),
        pltpu.SemaphoreType.DMA((nq,)),
        pltpu.SemaphoreType.DMA((nbuf,)),
    ]
    @functools.partial(pl.kernel, out_type=out_type, mesh=mesh,
                       scratch_types=scratch,
                       compiler_params=pltpu.CompilerParams(needs_layout_passes=False))
    def sc_agg(y_hbm, eidx_hbm, out_hbm, idx_v, rows_v, acc, gsem, ssem, isem, gsem2):
        cid = lax.axis_index("c")
        sid = lax.axis_index("s")

        def zrow(i, _):
            for j in range(_DH // 16):
                rows_v[0, i, pl.ds(j * 16, 16)] = jnp.zeros((16,), jnp.float32)
            return 0
        lax.fori_loop(0, k, zrow, 0)

        zbase = sid * zps
        for q in range(zps // k):
            pltpu.sync_copy(rows_v.at[0], acc.at[pl.ds(zbase + q * k, k)])
        ztail = zps % k
        if ztail:
            pltpu.sync_copy(rows_v.at[0].at[pl.ds(0, ztail)],
                            acc.at[pl.ds(zbase + (zps // k) * k, ztail)])

        plsc.subcore_barrier()

        cbase0 = sid * chunks

        def load_idx(g, q):
            pltpu.async_copy(eidx_hbm.at[cid, cbase0 + g], idx_v.at[q], isem.at[q])

        def wait_idx(g, q):
            pltpu.make_async_copy(eidx_hbm.at[cid, cbase0 + g], idx_v.at[q],
                                  isem.at[q]).wait()

        kh = (k // 2 + 7) // 8 * 8

        def start_gather(b, q):
            pltpu.async_copy(y_hbm.at[idx_v.at[q, 0, pl.ds(0, kh)]],
                             rows_v.at[b, pl.ds(0, kh)], gsem.at[b])
            pltpu.async_copy(y_hbm.at[idx_v.at[q, 0, pl.ds(kh, k - kh)]],
                             rows_v.at[b, pl.ds(kh, k - kh)], gsem2.at[b])

        def wait_gather(b, q):
            pltpu.make_async_copy(y_hbm.at[idx_v.at[q, 0, pl.ds(0, kh)]],
                                  rows_v.at[b, pl.ds(0, kh)], gsem.at[b]).wait()
            pltpu.make_async_copy(y_hbm.at[idx_v.at[q, 0, pl.ds(kh, k - kh)]],
                                  rows_v.at[b, pl.ds(kh, k - kh)], gsem2.at[b]).wait()

        def start_scatter(b, q):
            pltpu.async_copy(rows_v.at[b], acc.at[idx_v.at[q, 1]], ssem.at[b], add=True)

        def wait_scatter(b, q):
            pltpu.make_async_copy(rows_v.at[b], acc.at[idx_v.at[q, 1]],
                                  ssem.at[b]).wait()

        def step(g, u, first=False, do_load=True, do_gather=True):
            b, q = u % nbuf, u % nq
            wait_gather(b, q)
            start_scatter(b, q)
            if not first:
                wait_scatter((u - 1) % nbuf, (u - 1) % nq)
            if do_load:
                load_idx(g + nq - 1, (u - 1) % nq)
            if do_gather:
                gg = g + nbuf - 1
                ug = u + nbuf - 1
                wait_idx(gg, ug % nq)
                start_gather(ug % nbuf, ug % nq)

        for gg in range(nq - 1):
            load_idx(gg, gg)
        for gg in range(nbuf - 1):
            wait_idx(gg, gg)
            start_gather(gg, gg)

        for g in range(nq):
            step(g, g, first=(g == 0))

        def outer(go, _):
            for u in range(nq):
                step(go * nq + u, u)
            return 0
        lax.fori_loop(1, chunks // nq - 1, outer, 0)

        for g in range(chunks - nq, chunks):
            step(g, g, do_load=(g + nq - 1 < chunks),
                 do_gather=(g + nbuf - 1 < chunks))

        wait_scatter((chunks - 1) % nbuf, (chunks - 1) % nq)

        plsc.subcore_barrier()

        wbase = sid * rows_per_sub
        pltpu.sync_copy(acc.at[pl.ds(wbase, rows_per_sub)],
                        out_hbm.at[pl.ds(cid * _N + wbase, rows_per_sub)])

        @pl.when(sid == _NSUB - 1)
        def _():
            tail = _NSUB * rows_per_sub
            pltpu.sync_copy(acc.at[pl.ds(tail, _N - tail)],
                            out_hbm.at[pl.ds(cid * _N + tail, _N - tail)])


    return sc_agg


_ECNT = 163840


@functools.lru_cache(maxsize=None)
def _make_sc_cnt():
    per_tile = _ECNT // (_NCORE * _NSUB)
    mesh = plsc.VectorSubcoreMesh(
        core_axis_name="c", subcore_axis_name="s",
        num_cores=_NCORE, num_subcores=_NSUB)

    @functools.partial(
        pl.kernel,
        out_type=jax.ShapeDtypeStruct((_NCORE, _ACC1), jnp.float32),
        mesh=mesh,
        scratch_types=[
            pltpu.VMEM((per_tile,), jnp.int32),
            pltpu.VMEM((_ACC1,), jnp.float32),
            pltpu.VMEM((_NSUB, _RRED), jnp.float32),
            pltpu.VMEM((_RRED,), jnp.float32),
            pltpu.VMEM_SHARED((_NSUB, _ACC1), jnp.float32),
        ],
        compiler_params=pltpu.CompilerParams(needs_layout_passes=False))
    def sc_cnt(dst_hbm, cnt_hbm, dstb, hist, stg, red, stage):
        cid = lax.axis_index("c")
        sid = lax.axis_index("s")
        wid = sid * _NCORE + cid
        pltpu.sync_copy(dst_hbm.at[pl.ds(wid * per_tile, per_tile)], dstb)

        def hzero(i, _):
            hist[pl.ds(i * 16, 16)] = jnp.zeros((16,), jnp.float32)
            return 0
        lax.fori_loop(0, _ACC1 // 16, hzero, 0)

        ones16 = jnp.ones((16,), jnp.float32)

        def upd(i, _):
            di = dstb[pl.ds(i * 16, 16)]
            plsc.addupdate_scatter(hist, [di], ones16)
            return 0
        lax.fori_loop(0, per_tile // 16, upd, 0)

        pltpu.sync_copy(hist, stage.at[sid])
        plsc.subcore_barrier()

        pltpu.sync_copy(stage.at[:, pl.ds(sid * _RRED, _RRED)], stg)

        def red_t(t, _):
            for j in range(_RRED // 16):
                red[pl.ds(j * 16, 16)] = (red[pl.ds(j * 16, 16)]
                                          + stg[t, pl.ds(j * 16, 16)])
            return 0
        for j in range(_RRED // 16):
            red[pl.ds(j * 16, 16)] = stg[0, pl.ds(j * 16, 16)]
        lax.fori_loop(1, _NSUB, red_t, 0)
        pltpu.sync_copy(red, cnt_hbm.at[cid, pl.ds(sid * _RRED, _RRED)])

    return sc_cnt


def _mmA_body(x_ref, wl_ref, wr_ref, b_ref, y_ref, z_ref):
    xb = x_ref[...]
    dn = (((1,), (1,)), ((), ()))
    y = lax.dot_general(xb, wl_ref[...], dn, preferred_element_type=jnp.float32)
    z = lax.dot_general(xb, wr_ref[...], dn, preferred_element_type=jnp.float32)
    z_ref[...] = z + b_ref[...]
    y_ref[0] = y[:, :_DH]
    y_ref[1] = y[:, _DH:]


def _mmB_body(s_ref, c_ref, c2_ref, z_ref, wl_ref, wr_ref, b_ref, y_ref, z2_ref, inv_ref):
    inv = 1.0 / jnp.maximum(c_ref[...] + c2_ref[...], 1.0)
    h = jnp.maximum(
        jnp.concatenate([s_ref[0], s_ref[1]], axis=1) * inv + z_ref[...], 0.0)
    dn = (((1,), (1,)), ((), ()))
    y = lax.dot_general(h, wl_ref[...], dn, preferred_element_type=jnp.float32)
    z2 = lax.dot_general(h, wr_ref[...], dn, preferred_element_type=jnp.float32)
    z2_ref[...] = z2 + b_ref[...]
    y_ref[0] = y[:, :_DH]
    y_ref[1] = y[:, _DH:]
    inv_ref[...] = inv


def _mmC_body(s_ref, z_ref, inv_ref, o_ref):
    inv = inv_ref[...]
    o_ref[...] = jnp.concatenate([s_ref[0] * inv, s_ref[1] * inv], axis=1) + z_ref[...]


def _mmA(x, W1l, W1r, b1):
    grid = (_N // _BLK,)
    return pl.pallas_call(
        _mmA_body,
        grid=grid,
        in_specs=[
            pl.BlockSpec((_BLK, _D), lambda i: (i, 0)),
            pl.BlockSpec((_D, _D), lambda i: (0, 0)),
            pl.BlockSpec((_D, _D), lambda i: (0, 0)),
            pl.BlockSpec((1, _D), lambda i: (0, 0)),
        ],
        out_specs=[
            pl.BlockSpec((2, _BLK, _DH), lambda i: (0, i, 0)),
            pl.BlockSpec((_BLK, _D), lambda i: (i, 0)),
        ],
        out_shape=[
            jax.ShapeDtypeStruct((2, _N, _DH), jnp.float32),
            jax.ShapeDtypeStruct((_N, _D), jnp.float32),
        ],
    )(x, W1l, W1r, b1.reshape(1, _D))


def _mmB(S1, cnt, cnt2, z1, W2l, W2r, b2):
    grid = (_N // _BLK,)
    return pl.pallas_call(
        _mmB_body,
        grid=grid,
        in_specs=[
            pl.BlockSpec((2, _BLK, _DH), lambda i: (0, i, 0)),
            pl.BlockSpec((_BLK, 1), lambda i: (i, 0)),
            pl.BlockSpec((_BLK, 1), lambda i: (i, 0)),
            pl.BlockSpec((_BLK, _D), lambda i: (i, 0)),
            pl.BlockSpec((_D, _D), lambda i: (0, 0)),
            pl.BlockSpec((_D, _D), lambda i: (0, 0)),
            pl.BlockSpec((1, _D), lambda i: (0, 0)),
        ],
        out_specs=[
            pl.BlockSpec((2, _BLK, _DH), lambda i: (0, i, 0)),
            pl.BlockSpec((_BLK, _D), lambda i: (i, 0)),
            pl.BlockSpec((_BLK, 1), lambda i: (i, 0)),
        ],
        out_shape=[
            jax.ShapeDtypeStruct((2, _N, _DH), jnp.float32),
            jax.ShapeDtypeStruct((_N, _D), jnp.float32),
            jax.ShapeDtypeStruct((_N, 1), jnp.float32),
        ],
    )(S1, cnt, cnt2, z1, W2l, W2r, b2.reshape(1, _D))


def _mmC(S2, z2, inv):
    grid = (_N // _BLK,)
    return pl.pallas_call(
        _mmC_body,
        grid=grid,
        in_specs=[
            pl.BlockSpec((2, _BLK, _DH), lambda i: (0, i, 0)),
            pl.BlockSpec((_BLK, _D), lambda i: (i, 0)),
            pl.BlockSpec((_BLK, 1), lambda i: (i, 0)),
        ],
        out_specs=pl.BlockSpec((_BLK, _D), lambda i: (i, 0)),
        out_shape=jax.ShapeDtypeStruct((_N, _D), jnp.float32),
    )(S2, z2, inv)


def _edge_chunks(edge_index, epad, k):
    pad = epad - _E
    src = jnp.concatenate([edge_index[0], jnp.zeros((pad,), jnp.int32)]).reshape(-1, k)
    dst = jnp.concatenate([edge_index[1], jnp.full((pad,), _N, jnp.int32)]).reshape(-1, k)
    lo = jnp.stack([src, dst], axis=1)
    hi = jnp.stack([src + _N, dst], axis=1)
    return jnp.stack([lo, hi])


def kernel(x, edge_index, W1l, b1, W1r, W2l, b2, W2r):
    e2 = _edge_chunks(edge_index, _NSUB * 84 * 120, 120)
    dst_p = jnp.concatenate(
        [edge_index[1], jnp.full((_ECNT - _E,), _N, jnp.int32)])

    cnt01 = _make_sc_cnt()(dst_p)
    y1, z1 = _mmA(x, W1l, W1r, b1)
    (S1,) = _make_sc_agg(84, 3, _ACC2, 120)(y1.reshape(_NCORE * _N, _DH), e2)
    cnt = cnt01[0, :_N].reshape(_N, 1)
    cnt2 = cnt01[1, :_N].reshape(_N, 1)
    y2, z2, inv = _mmB(S1.reshape(_NCORE, _N, _DH), cnt, cnt2, z1, W2l, W2r, b2)
    (S2,) = _make_sc_agg(84, 3, _ACC2, 120)(y2.reshape(_NCORE * _N, _DH), e2)
    return _mmC(S2.reshape(_NCORE, _N, _DH), z2, inv)
